# baseline (speedup 1.0000x reference)
import functools

import jax
import jax.numpy as jnp
from jax import lax
from jax.experimental import pallas as pl
from jax.experimental.pallas import tpu as pltpu
from jax.experimental.pallas import tpu_sc as plsc

_B = 4096
_V = 100000
_D = 64
_W = 5
_NEG = 5
_NW = 32
_BPW = _B // _NW
_NF = 2 * _W + 2 + 2 * _NEG
_NG = _BPW // 16
_GCOLS = list(range(2 * _W)) + [2 * _W + 1] + list(range(2 * _W + 2,
                                                         2 * _W + 2 + _NEG))
_NT = 1 + _NEG


def _sc_body(data_hbm, emb0_hbm, emb1_hbm, out_hbm,
             d22, idxs, pv, acc, rb0, rb1, rb2, tb,
             linv_v, mask_v, res_v,
             s_r0, s_r1, s_r2, s_t):
    wid = lax.axis_index("s") * 2 + lax.axis_index("c")
    base = wid * _BPW
    lane = lax.iota(jnp.int32, 16)

    pltpu.sync_copy(data_hbm.at[pl.ds(base, _BPW)], d22)

    one_i = jnp.full((16,), 1, jnp.int32)
    for f, col in enumerate(_GCOLS):
        cvec = jnp.full((16,), col, jnp.int32)
        for g in range(_NG):
            v = plsc.load_gather(d22, [lane + g * 16, cvec])
            idxs[f, pl.ds(g * 16, 16)] = lax.shift_right_logical(v, one_i)
            pv[f, pl.ds(g * 16, 16)] = lax.shift_left(
                lax.bitwise_and(v, one_i), jnp.full((16,), 6, jnp.int32))

    ring = [rb0, rb1, rb2]
    ring_sems = [s_r0, s_r1, s_r2]

    cps = {
        j: pltpu.async_copy(emb0_hbm.at[idxs.at[j]], ring[j], ring_sems[j])
        for j in range(3)
    }
    ct = pltpu.async_copy(emb1_hbm.at[idxs.at[2 * _W]], tb, s_t)

    lcol = jnp.full((16,), 2 * _W, jnp.int32)
    fone = jnp.full((16,), 1.0, jnp.float32)
    for g in range(_NG):
        lv = plsc.load_gather(d22, [lane + g * 16, lcol])
        linv_v[pl.ds(g * 16, 16)] = fone / lv.astype(jnp.float32)
    for n in range(_NEG):
        mcol = jnp.full((16,), 2 * _W + 2 + _NEG + n, jnp.int32)
        for g in range(_NG):
            mv = plsc.load_gather(d22, [lane + g * 16, mcol])
            mask_v[pl.ds(n * _BPW + g * 16, 16)] = mv.astype(jnp.float32)

    bidxs = [lane + g * 16 for g in range(_NG)]
    for j in range(2 * _W):
        slot = j % 3
        buf = ring[slot]
        cps[j].wait()
        poffs = [pv[j, pl.ds(g * 16, 16)] for g in range(_NG)]

        def acc_body(d, _, buf=buf, poffs=poffs, j=j):
            dvec = jnp.full((16,), d, jnp.int32)
            for g in range(_NG):
                val = plsc.load_gather(buf, [bidxs[g], poffs[g] + dvec])
                if j == 0:
                    acc[d, pl.ds(g * 16, 16)] = val
                else:
                    plsc.addupdate(acc.at[d, pl.ds(g * 16, 16)], val)
            return 0

        lax.fori_loop(0, _D, acc_body, 0)
        nxt = j + 3
        if nxt < 2 * _W:
            cps[nxt] = pltpu.async_copy(
                emb0_hbm.at[idxs.at[nxt]], buf, ring_sems[slot])

    tbufs = [tb, rb0, rb1, rb2, tb, rb0]
    tsems = [s_t, s_r0, s_r1, s_r2, s_t, s_r0]
    cts = {0: ct}
    zeros = jnp.zeros((16,), jnp.float32)
    for t in range(_NT):
        f = 2 * _W + t
        buf = tbufs[t]
        cts[t].wait()
        if t + 1 < _NT:
            cts[t + 1] = pltpu.async_copy(
                emb1_hbm.at[idxs.at[f + 1]], tbufs[t + 1], tsems[t + 1])
        poffs = [pv[f, pl.ds(g * 16, 16)] for g in range(_NG)]

        def dot_step(d, carry, buf=buf, poffs=poffs):
            dvec = jnp.full((16,), d, jnp.int32)
            out = []
            for g in range(_NG):
                av = acc[d, pl.ds(g * 16, 16)]
                tv = plsc.load_gather(buf, [bidxs[g], poffs[g] + dvec])
                out.append(carry[g] + av * tv)
            return tuple(out)

        res = lax.fori_loop(0, _D, dot_step, (zeros,) * _NG)
        for g in range(_NG):
            r = res[g] * linv_v[pl.ds(g * 16, 16)]
            if t > 0:
                r = r * mask_v[pl.ds((t - 1) * _BPW + g * 16, 16)]
            res_v[pl.ds(t * _BPW + g * 16, 16)] = r

    pltpu.sync_copy(res_v.at[pl.ds(0, _BPW)], out_hbm.at[0, wid])
    for n in range(_NEG):
        pltpu.sync_copy(res_v.at[pl.ds((1 + n) * _BPW, _BPW)],
                        out_hbm.at[1 + n, wid])
        pltpu.sync_copy(mask_v.at[pl.ds(n * _BPW, _BPW)],
                        out_hbm.at[1 + _NEG + n, wid])


_sc_kernel = functools.partial(
    pl.kernel,
    out_type=jax.ShapeDtypeStruct((1 + 2 * _NEG, _NW, _BPW), jnp.float32),
    mesh=plsc.VectorSubcoreMesh(core_axis_name="c", subcore_axis_name="s"),
    compiler_params=pltpu.CompilerParams(needs_layout_passes=False),
    scratch_types=[
        pltpu.VMEM((_BPW, _NF), jnp.int32),
        pltpu.VMEM((2 * _W + _NT, _BPW), jnp.int32),
        pltpu.VMEM((2 * _W + _NT, _BPW), jnp.int32),
        pltpu.VMEM((_D, _BPW), jnp.float32),
        *[pltpu.VMEM((_BPW, 2 * _D), jnp.float32) for _ in range(4)],
        pltpu.VMEM((_BPW,), jnp.float32),
        pltpu.VMEM((_NEG * _BPW,), jnp.float32),
        pltpu.VMEM((_NT * _BPW,), jnp.float32),
        *[pltpu.SemaphoreType.DMA for _ in range(4)],
    ],
)(_sc_body)


def _loss_body(s_ref, o_ref):
    x = jnp.clip(s_ref[0], -10.0, 10.0)
    total = jnp.sum(jnp.log(1.0 + jnp.exp(-x)))
    for n in range(_NEG):
        z = jnp.clip(-s_ref[1 + n], -10.0, 10.0)
        total = total + jnp.sum(jnp.log(1.0 + jnp.exp(-z))
                                * s_ref[1 + _NEG + n])
    o_ref[...] = jnp.reshape(total, (1, 1))


def kernel(data, emb0, emb1):
    e0 = lax.slice(emb0, (0, 0), (_V, _D)).reshape(_V // 2, 2 * _D)
    e1 = emb1.reshape(_V // 2, 2 * _D)
    raw = _sc_kernel(data, e0, e1)
    loss = pl.pallas_call(
        _loss_body,
        out_shape=jax.ShapeDtypeStruct((1, 1), jnp.float32),
    )(raw)
    return loss[0, 0]
